# baseline (speedup 1.0000x reference)
import functools

import jax
import jax.numpy as jnp
from jax import lax
from jax.experimental import pallas as pl
from jax.experimental.pallas import tpu as pltpu
from jax.experimental.pallas import tpu_sc as plsc

N = 10000
E = 320000
D = 128
NC = 2
NS = 16
NW = NC * NS
EPW = E // NW
CH = 80
NCH = EPW // CH
NP = 10112
RPT = NP // NS
NBUF = 4
SS = 2
ISLOTS = 2 * NBUF
NGRP = NCH // ISLOTS
NTAIL = NCH - NGRP * ISLOTS
DEG_K = 25
DW = 16

_MESH = plsc.VectorSubcoreMesh(core_axis_name="c", subcore_axis_name="s")


def _deg_body(dst_hbm, zero_hbm, ones_hbm, out_hbm, dst_v, ones_v, sem, acc):
    cid = lax.axis_index("c")
    sid = lax.axis_index("s")
    pltpu.sync_copy(dst_hbm.at[cid, sid], dst_v)
    pltpu.sync_copy(ones_hbm, ones_v)
    pltpu.sync_copy(zero_hbm.at[pl.ds(sid * RPT, RPT)], acc.at[pl.ds(sid * RPT, RPT)])
    plsc.subcore_barrier()

    def grp(g, carry):
        def fire(j, c):
            pltpu.async_copy(ones_v, acc.at[dst_v.at[g * DEG_K + j]], sem, add=True)
            return c

        lax.fori_loop(0, DEG_K, fire, 0)

        def drain(j, c):
            pltpu.make_async_copy(ones_v, acc.at[dst_v.at[0]], sem).wait()
            return c

        lax.fori_loop(0, DEG_K, drain, 0)
        return carry

    lax.fori_loop(0, NCH // DEG_K, grp, 0)
    plsc.subcore_barrier()
    pltpu.sync_copy(acc.at[pl.ds(sid * RPT, RPT)],
                    out_hbm.at[cid, pl.ds(sid * RPT, RPT)])


_deg_call = pl.kernel(
    _deg_body,
    out_type=jax.ShapeDtypeStruct((NC, NP, DW), jnp.float32),
    mesh=_MESH,
    scratch_types=[
        pltpu.VMEM((NCH, CH), jnp.int32),
        pltpu.VMEM((CH, DW), jnp.float32),
        pltpu.SemaphoreType.DMA,
        pltpu.VMEM_SHARED((NP, DW), jnp.float32),
    ],
    compiler_params=pltpu.CompilerParams(use_tc_tiling_on_sc=False),
)


def _spmm_body(h_hbm, src_hbm, dst_hbm, out_hbm, rows, sbuf, dbuf,
               gsem, isem, dsem, ssem, acc):
    cid = lax.axis_index("c")
    sid = lax.axis_index("s")
    pltpu.sync_copy(h_hbm.at[pl.ds(sid * RPT, RPT)], acc.at[pl.ds(sid * RPT, RPT)])
    plsc.subcore_barrier()

    def fire_idx(j, islot):
        pltpu.async_copy(src_hbm.at[cid, sid, pl.ds(j, 1)],
                         sbuf.at[pl.ds(islot, 1)], isem.at[islot])
        pltpu.async_copy(dst_hbm.at[cid, sid, pl.ds(j, 1)],
                         dbuf.at[pl.ds(islot, 1)], dsem.at[islot])

    def wait_idx(islot, sem):
        pltpu.make_async_copy(src_hbm.at[cid, sid, pl.ds(0, 1)],
                              sbuf.at[pl.ds(islot, 1)], sem.at[islot]).wait()

    def fire_gather(islot, rb):
        pltpu.async_copy(h_hbm.at[sbuf.at[islot]], rows.at[rb], gsem.at[rb])

    def wait_scatter(rb):
        pltpu.make_async_copy(rows.at[rb], acc.at[dbuf.at[0]],
                              ssem.at[rb]).wait()

    def step(j, rb, islot, fire_idx_next, regather, wait_prev_scatter):
        if fire_idx_next:
            fire_idx(j + NBUF, (islot + NBUF) % ISLOTS)
        pltpu.make_async_copy(h_hbm.at[sbuf.at[islot]], rows.at[rb],
                              gsem.at[rb]).wait()
        wait_idx(islot, dsem)
        pltpu.async_copy(rows.at[rb], acc.at[dbuf.at[islot]], ssem.at[rb],
                         add=True)
        if regather:
            mrb = (rb + NBUF - SS) % NBUF
            mislot = (islot + NBUF - SS) % ISLOTS
            if wait_prev_scatter:
                wait_scatter(mrb)
            wait_idx(mislot, isem)
            fire_gather(mislot, mrb)

    for b in range(NBUF):
        fire_idx(b, b)
    for b in range(NBUF - SS):
        wait_idx(b, isem)
        fire_gather(b, b)

    for s in range(ISLOTS):
        step(s, s % NBUF, s, True, True, s >= SS)

    def grp(g, carry):
        for s in range(ISLOTS):
            step(g * ISLOTS + s, s % NBUF, s, True, True, True)
        return carry

    lax.fori_loop(1, NGRP, grp, 0)
    for t in range(NTAIL):
        j = NGRP * ISLOTS + t
        step(j, j % NBUF, j % ISLOTS,
             j + NBUF < NCH, j + NBUF - SS < NCH, True)
    for k in range(NBUF):
        wait_scatter((NCH - 1 - k) % NBUF)
    plsc.subcore_barrier()
    pltpu.sync_copy(acc.at[pl.ds(sid * RPT, RPT)],
                    out_hbm.at[cid, pl.ds(sid * RPT, RPT)])


_spmm_call = pl.kernel(
    _spmm_body,
    out_type=jax.ShapeDtypeStruct((NC, NP, D), jnp.float32),
    mesh=_MESH,
    scratch_types=[
        pltpu.VMEM((NBUF, CH, D), jnp.float32),
        pltpu.VMEM((ISLOTS, CH), jnp.int32),
        pltpu.VMEM((ISLOTS, CH), jnp.int32),
        pltpu.SemaphoreType.DMA((NBUF,)),
        pltpu.SemaphoreType.DMA((ISLOTS,)),
        pltpu.SemaphoreType.DMA((ISLOTS,)),
        pltpu.SemaphoreType.DMA((NBUF,)),
        pltpu.VMEM_SHARED((NP, D), jnp.float32),
    ],
)


def _dinv(p0, p1):
    deg = p0[:N, 0] + p1[:N, 0] + 1.0
    return lax.rsqrt(deg)


def _tc_first(x_ref, p0_ref, p1_ref, w_ref, h_out):
    x = x_ref[...]
    mean = jnp.mean(x, axis=0)
    xc = x - mean
    var = jnp.mean(xc * xc, axis=0)
    xn = xc * lax.rsqrt(var + 1e-5)
    s = jnp.sign(xn)
    h = jnp.dot(s, w_ref[...], preferred_element_type=jnp.float32)
    dinv = _dinv(p0_ref[...], p1_ref[...])
    h_out[0:N, :] = h * dinv[:, None]
    h_out[N:NP, :] = jnp.zeros((NP - N, D), jnp.float32)


def _tc_mid(p_ref, hprev_ref, p0_ref, p1_ref, w_ref, b_ref, m2_ref, h_out):
    dinv = _dinv(p0_ref[...], p1_ref[...])
    y = p_ref[0, 0:N, :] + p_ref[1, 0:N, :] - hprev_ref[0:N, :]
    xcur = y * dinv[:, None] + b_ref[...][None, :]
    xcur = xcur * m2_ref[...]
    s = jnp.sign(xcur)
    h = jnp.dot(s, w_ref[...], preferred_element_type=jnp.float32)
    h_out[0:N, :] = h * dinv[:, None]
    h_out[N:NP, :] = jnp.zeros((NP - N, D), jnp.float32)


def _tc_last(p_ref, hprev_ref, p0_ref, p1_ref, b_ref, out_ref):
    dinv = _dinv(p0_ref[...], p1_ref[...])
    y = p_ref[0, 0:N, :] + p_ref[1, 0:N, :] - hprev_ref[0:N, :]
    xcur = y * dinv[:, None] + b_ref[...][None, :]
    mx = jnp.max(xcur, axis=1, keepdims=True)
    sh = xcur - mx
    lse = jnp.log(jnp.sum(jnp.exp(sh), axis=1, keepdims=True))
    out_ref[...] = sh - lse


_f32 = jnp.float32
_tc_first_call = pl.pallas_call(
    _tc_first, out_shape=jax.ShapeDtypeStruct((NP, D), _f32))
_tc_mid_call = pl.pallas_call(
    _tc_mid, out_shape=jax.ShapeDtypeStruct((NP, D), _f32))
_tc_last_call = pl.pallas_call(
    _tc_last, out_shape=jax.ShapeDtypeStruct((N, D), _f32))


def kernel(x, edge_index, W0, b0, W1, b1, W2, b2):
    src = edge_index[0].astype(jnp.int32).reshape(NC, NS, NCH, CH)
    dst = edge_index[1].astype(jnp.int32).reshape(NC, NS, NCH, CH)

    zerosd = jnp.zeros((NP, DW), _f32)
    onesd = jnp.ones((CH, DW), _f32)
    degp = _deg_call(dst, zerosd, onesd)
    p0d = degp[0]
    p1d = degp[1]

    m0 = jax.random.bernoulli(
        jax.random.fold_in(jax.random.key(42), 0), 0.5, (N, D)).astype(_f32) * 2.0
    m1 = jax.random.bernoulli(
        jax.random.fold_in(jax.random.key(42), 1), 0.5, (N, D)).astype(_f32) * 2.0

    h0 = _tc_first_call(x, p0d, p1d, W0)
    p = _spmm_call(h0, src, dst)
    h1 = _tc_mid_call(p, h0, p0d, p1d, W1, b0, m0)
    p = _spmm_call(h1, src, dst)
    h2 = _tc_mid_call(p, h1, p0d, p1d, W2, b1, m1)
    p = _spmm_call(h2, src, dst)
    return _tc_last_call(p, h2, p0d, p1d, b2)


# speedup vs baseline: 1.1468x; 1.1468x over previous
import functools

import jax
import jax.numpy as jnp
from jax import lax
from jax.experimental import pallas as pl
from jax.experimental.pallas import tpu as pltpu
from jax.experimental.pallas import tpu_sc as plsc

N = 10000
E = 320000
D = 128
NC = 2
NS = 16
NW = NC * NS
EPW = E // NW
CH = 80
NCH = EPW // CH
NP = 10112
RPT = NP // NS
NBUF = 4
SS = 2
ISLOTS = 2 * NBUF
NGRP = NCH // ISLOTS
NTAIL = NCH - NGRP * ISLOTS
DEG_K = 25
DW = 16

_MESH = plsc.VectorSubcoreMesh(core_axis_name="c", subcore_axis_name="s")


def _deg_body(dst_hbm, zero_hbm, ones_hbm, out_hbm, dst_v, ones_v, sem, acc):
    cid = lax.axis_index("c")
    sid = lax.axis_index("s")
    pltpu.sync_copy(dst_hbm.at[cid, sid], dst_v)
    pltpu.sync_copy(ones_hbm, ones_v)
    pltpu.sync_copy(zero_hbm.at[pl.ds(sid * RPT, RPT)], acc.at[pl.ds(sid * RPT, RPT)])
    plsc.subcore_barrier()

    def grp(g, carry):
        def fire(j, c):
            pltpu.async_copy(ones_v, acc.at[dst_v.at[g * DEG_K + j]], sem, add=True)
            return c

        lax.fori_loop(0, DEG_K, fire, 0)

        def drain(j, c):
            pltpu.make_async_copy(ones_v, acc.at[dst_v.at[0]], sem).wait()
            return c

        lax.fori_loop(0, DEG_K, drain, 0)
        return carry

    lax.fori_loop(0, NCH // DEG_K, grp, 0)
    plsc.subcore_barrier()
    pltpu.sync_copy(acc.at[pl.ds(sid * RPT, RPT)],
                    out_hbm.at[cid, pl.ds(sid * RPT, RPT)])


_deg_call = pl.kernel(
    _deg_body,
    out_type=jax.ShapeDtypeStruct((NC, NP, DW), jnp.float32),
    mesh=_MESH,
    scratch_types=[
        pltpu.VMEM((NCH, CH), jnp.int32),
        pltpu.VMEM((CH, DW), jnp.float32),
        pltpu.SemaphoreType.DMA,
        pltpu.VMEM_SHARED((NP, DW), jnp.float32),
    ],
    compiler_params=pltpu.CompilerParams(use_tc_tiling_on_sc=False),
)


def _spmm_body(h_hbm, src_hbm, dst_hbm, out_hbm, rows, sbuf, dbuf,
               gsem, isem, dsem, acc):
    cid = lax.axis_index("c")
    sid = lax.axis_index("s")
    pltpu.sync_copy(h_hbm.at[pl.ds(sid * RPT, RPT)], acc.at[pl.ds(sid * RPT, RPT)])
    plsc.subcore_barrier()

    def fire_idx(j, islot):
        pltpu.async_copy(src_hbm.at[cid, sid, pl.ds(j, 1)],
                         sbuf.at[pl.ds(islot, 1)], isem.at[islot])
        pltpu.async_copy(dst_hbm.at[cid, sid, pl.ds(j, 1)],
                         dbuf.at[pl.ds(islot, 1)], dsem.at[islot])

    def wait_idx(islot, sem):
        pltpu.make_async_copy(src_hbm.at[cid, sid, pl.ds(0, 1)],
                              sbuf.at[pl.ds(islot, 1)], sem.at[islot]).wait()

    def fire_gather(islot, rb):
        pltpu.async_copy(h_hbm.at[sbuf.at[islot]], rows.at[rb], gsem.at[rb])

    def step(j, rb, islot, fire_next):
        if fire_next:
            fire_idx(j + NBUF, (islot + NBUF) % ISLOTS)
        pltpu.make_async_copy(h_hbm.at[sbuf.at[islot]], rows.at[rb],
                              gsem.at[rb]).wait()
        wait_idx(islot, dsem)
        pltpu.sync_copy(rows.at[rb], acc.at[dbuf.at[islot]], add=True)
        if fire_next:
            wait_idx((islot + NBUF) % ISLOTS, isem)
            fire_gather((islot + NBUF) % ISLOTS, rb)

    for b in range(NBUF):
        fire_idx(b, b)
    for b in range(NBUF):
        wait_idx(b, isem)
        fire_gather(b, b)

    def grp(g, carry):
        for s in range(ISLOTS):
            step(g * ISLOTS + s, s % NBUF, s, True)
        return carry

    lax.fori_loop(0, NGRP, grp, 0)
    for t in range(NTAIL):
        j = NGRP * ISLOTS + t
        step(j, t % NBUF, t, t + NBUF < NTAIL)
    plsc.subcore_barrier()
    pltpu.sync_copy(acc.at[pl.ds(sid * RPT, RPT)],
                    out_hbm.at[cid, pl.ds(sid * RPT, RPT)])


_spmm_call = pl.kernel(
    _spmm_body,
    out_type=jax.ShapeDtypeStruct((NC, NP, D), jnp.float32),
    mesh=_MESH,
    scratch_types=[
        pltpu.VMEM((NBUF, CH, D), jnp.float32),
        pltpu.VMEM((ISLOTS, CH), jnp.int32),
        pltpu.VMEM((ISLOTS, CH), jnp.int32),
        pltpu.SemaphoreType.DMA((NBUF,)),
        pltpu.SemaphoreType.DMA((ISLOTS,)),
        pltpu.SemaphoreType.DMA((ISLOTS,)),
        pltpu.VMEM_SHARED((NP, D), jnp.float32),
    ],
)


def _dinv(p0, p1):
    deg = p0[:N, 0] + p1[:N, 0] + 1.0
    return lax.rsqrt(deg)


def _tc_first(x_ref, p0_ref, p1_ref, w_ref, h_out):
    x = x_ref[...]
    mean = jnp.mean(x, axis=0)
    xc = x - mean
    var = jnp.mean(xc * xc, axis=0)
    xn = xc * lax.rsqrt(var + 1e-5)
    s = jnp.sign(xn)
    h = jnp.dot(s, w_ref[...], preferred_element_type=jnp.float32)
    dinv = _dinv(p0_ref[...], p1_ref[...])
    h_out[0:N, :] = h * dinv[:, None]
    h_out[N:NP, :] = jnp.zeros((NP - N, D), jnp.float32)


def _tc_mid(p_ref, hprev_ref, p0_ref, p1_ref, w_ref, b_ref, m2_ref, h_out):
    dinv = _dinv(p0_ref[...], p1_ref[...])
    y = p_ref[0, 0:N, :] + p_ref[1, 0:N, :] - hprev_ref[0:N, :]
    xcur = y * dinv[:, None] + b_ref[...][None, :]
    xcur = xcur * m2_ref[...]
    s = jnp.sign(xcur)
    h = jnp.dot(s, w_ref[...], preferred_element_type=jnp.float32)
    h_out[0:N, :] = h * dinv[:, None]
    h_out[N:NP, :] = jnp.zeros((NP - N, D), jnp.float32)


def _tc_last(p_ref, hprev_ref, p0_ref, p1_ref, b_ref, out_ref):
    dinv = _dinv(p0_ref[...], p1_ref[...])
    y = p_ref[0, 0:N, :] + p_ref[1, 0:N, :] - hprev_ref[0:N, :]
    xcur = y * dinv[:, None] + b_ref[...][None, :]
    mx = jnp.max(xcur, axis=1, keepdims=True)
    sh = xcur - mx
    lse = jnp.log(jnp.sum(jnp.exp(sh), axis=1, keepdims=True))
    out_ref[...] = sh - lse


_f32 = jnp.float32
_tc_first_call = pl.pallas_call(
    _tc_first, out_shape=jax.ShapeDtypeStruct((NP, D), _f32))
_tc_mid_call = pl.pallas_call(
    _tc_mid, out_shape=jax.ShapeDtypeStruct((NP, D), _f32))
_tc_last_call = pl.pallas_call(
    _tc_last, out_shape=jax.ShapeDtypeStruct((N, D), _f32))


def kernel(x, edge_index, W0, b0, W1, b1, W2, b2):
    src = edge_index[0].astype(jnp.int32).reshape(NC, NS, NCH, CH)
    dst = edge_index[1].astype(jnp.int32).reshape(NC, NS, NCH, CH)

    zerosd = jnp.zeros((NP, DW), _f32)
    onesd = jnp.ones((CH, DW), _f32)
    degp = _deg_call(dst, zerosd, onesd)
    p0d = degp[0]
    p1d = degp[1]

    m0 = jax.random.bernoulli(
        jax.random.fold_in(jax.random.key(42), 0), 0.5, (N, D)).astype(_f32) * 2.0
    m1 = jax.random.bernoulli(
        jax.random.fold_in(jax.random.key(42), 1), 0.5, (N, D)).astype(_f32) * 2.0

    h0 = _tc_first_call(x, p0d, p1d, W0)
    p = _spmm_call(h0, src, dst)
    h1 = _tc_mid_call(p, h0, p0d, p1d, W1, b0, m0)
    p = _spmm_call(h1, src, dst)
    h2 = _tc_mid_call(p, h1, p0d, p1d, W2, b1, m1)
    p = _spmm_call(h2, src, dst)
    return _tc_last_call(p, h2, p0d, p1d, b2)


# speedup vs baseline: 1.1499x; 1.0027x over previous
import functools

import jax
import jax.numpy as jnp
from jax import lax
from jax.experimental import pallas as pl
from jax.experimental.pallas import tpu as pltpu
from jax.experimental.pallas import tpu_sc as plsc

N = 10000
E = 320000
D = 128
NC = 2
NS = 16
NW = NC * NS
EPW = E // NW
CH = 100
NCH = EPW // CH
NP = 10112
RPT = NP // NS
NBUF = 3
SS = 2
ISLOTS = 2 * NBUF
NGRP = NCH // ISLOTS
NTAIL = NCH - NGRP * ISLOTS
DEG_K = 25
DW = 16

_MESH = plsc.VectorSubcoreMesh(core_axis_name="c", subcore_axis_name="s")


def _deg_body(dst_hbm, zero_hbm, ones_hbm, out_hbm, dst_v, ones_v, sem, acc):
    cid = lax.axis_index("c")
    sid = lax.axis_index("s")
    pltpu.sync_copy(dst_hbm.at[cid, sid], dst_v)
    pltpu.sync_copy(ones_hbm, ones_v)
    pltpu.sync_copy(zero_hbm.at[pl.ds(sid * RPT, RPT)], acc.at[pl.ds(sid * RPT, RPT)])
    plsc.subcore_barrier()

    def grp(g, carry):
        def fire(j, c):
            pltpu.async_copy(ones_v, acc.at[dst_v.at[g * DEG_K + j]], sem, add=True)
            return c

        lax.fori_loop(0, DEG_K, fire, 0)

        def drain(j, c):
            pltpu.make_async_copy(ones_v, acc.at[dst_v.at[0]], sem).wait()
            return c

        lax.fori_loop(0, DEG_K, drain, 0)
        return carry

    lax.fori_loop(0, NCH // DEG_K, grp, 0)
    plsc.subcore_barrier()
    pltpu.sync_copy(acc.at[pl.ds(sid * RPT, RPT)],
                    out_hbm.at[cid, pl.ds(sid * RPT, RPT)])


_deg_call = pl.kernel(
    _deg_body,
    out_type=jax.ShapeDtypeStruct((NC, NP, DW), jnp.float32),
    mesh=_MESH,
    scratch_types=[
        pltpu.VMEM((NCH, CH), jnp.int32),
        pltpu.VMEM((CH, DW), jnp.float32),
        pltpu.SemaphoreType.DMA,
        pltpu.VMEM_SHARED((NP, DW), jnp.float32),
    ],
    compiler_params=pltpu.CompilerParams(use_tc_tiling_on_sc=False),
)


def _spmm_body(h_hbm, src_hbm, dst_hbm, out_hbm, rows, sbuf, dbuf,
               gsem, isem, dsem, acc):
    cid = lax.axis_index("c")
    sid = lax.axis_index("s")
    pltpu.sync_copy(h_hbm.at[pl.ds(sid * RPT, RPT)], acc.at[pl.ds(sid * RPT, RPT)])
    plsc.subcore_barrier()

    def fire_idx(j, islot):
        pltpu.async_copy(src_hbm.at[cid, sid, pl.ds(j, 1)],
                         sbuf.at[pl.ds(islot, 1)], isem.at[islot])
        pltpu.async_copy(dst_hbm.at[cid, sid, pl.ds(j, 1)],
                         dbuf.at[pl.ds(islot, 1)], dsem.at[islot])

    def wait_idx(islot, sem):
        pltpu.make_async_copy(src_hbm.at[cid, sid, pl.ds(0, 1)],
                              sbuf.at[pl.ds(islot, 1)], sem.at[islot]).wait()

    def fire_gather(islot, rb):
        pltpu.async_copy(h_hbm.at[sbuf.at[islot]], rows.at[rb], gsem.at[rb])

    def step(j, rb, islot, fire_next):
        if fire_next:
            fire_idx(j + NBUF, (islot + NBUF) % ISLOTS)
        pltpu.make_async_copy(h_hbm.at[sbuf.at[islot]], rows.at[rb],
                              gsem.at[rb]).wait()
        wait_idx(islot, dsem)
        pltpu.sync_copy(rows.at[rb], acc.at[dbuf.at[islot]], add=True)
        if fire_next:
            wait_idx((islot + NBUF) % ISLOTS, isem)
            fire_gather((islot + NBUF) % ISLOTS, rb)

    for b in range(NBUF):
        fire_idx(b, b)
    for b in range(NBUF):
        wait_idx(b, isem)
        fire_gather(b, b)

    def grp(g, carry):
        for s in range(ISLOTS):
            step(g * ISLOTS + s, s % NBUF, s, True)
        return carry

    lax.fori_loop(0, NGRP, grp, 0)
    for t in range(NTAIL):
        j = NGRP * ISLOTS + t
        step(j, t % NBUF, t, t + NBUF < NTAIL)
    plsc.subcore_barrier()
    pltpu.sync_copy(acc.at[pl.ds(sid * RPT, RPT)],
                    out_hbm.at[cid, pl.ds(sid * RPT, RPT)])


_spmm_call = pl.kernel(
    _spmm_body,
    out_type=jax.ShapeDtypeStruct((NC, NP, D), jnp.float32),
    mesh=_MESH,
    scratch_types=[
        pltpu.VMEM((NBUF, CH, D), jnp.float32),
        pltpu.VMEM((ISLOTS, CH), jnp.int32),
        pltpu.VMEM((ISLOTS, CH), jnp.int32),
        pltpu.SemaphoreType.DMA((NBUF,)),
        pltpu.SemaphoreType.DMA((ISLOTS,)),
        pltpu.SemaphoreType.DMA((ISLOTS,)),
        pltpu.VMEM_SHARED((NP, D), jnp.float32),
    ],
)


def _dinv(p0, p1):
    deg = p0[:N, 0] + p1[:N, 0] + 1.0
    return lax.rsqrt(deg)


def _tc_first(x_ref, p0_ref, p1_ref, w_ref, h_out):
    x = x_ref[...]
    mean = jnp.mean(x, axis=0)
    xc = x - mean
    var = jnp.mean(xc * xc, axis=0)
    xn = xc * lax.rsqrt(var + 1e-5)
    s = jnp.sign(xn)
    h = jnp.dot(s, w_ref[...], preferred_element_type=jnp.float32)
    dinv = _dinv(p0_ref[...], p1_ref[...])
    h_out[0:N, :] = h * dinv[:, None]
    h_out[N:NP, :] = jnp.zeros((NP - N, D), jnp.float32)


def _tc_mid(p_ref, hprev_ref, p0_ref, p1_ref, w_ref, b_ref, m2_ref, h_out):
    dinv = _dinv(p0_ref[...], p1_ref[...])
    y = p_ref[0, 0:N, :] + p_ref[1, 0:N, :] - hprev_ref[0:N, :]
    xcur = y * dinv[:, None] + b_ref[...][None, :]
    xcur = xcur * (m2_ref[...].astype(jnp.float32) * 2.0)
    s = jnp.sign(xcur)
    h = jnp.dot(s, w_ref[...], preferred_element_type=jnp.float32)
    h_out[0:N, :] = h * dinv[:, None]
    h_out[N:NP, :] = jnp.zeros((NP - N, D), jnp.float32)


def _tc_last(p_ref, hprev_ref, p0_ref, p1_ref, b_ref, out_ref):
    dinv = _dinv(p0_ref[...], p1_ref[...])
    y = p_ref[0, 0:N, :] + p_ref[1, 0:N, :] - hprev_ref[0:N, :]
    xcur = y * dinv[:, None] + b_ref[...][None, :]
    mx = jnp.max(xcur, axis=1, keepdims=True)
    sh = xcur - mx
    lse = jnp.log(jnp.sum(jnp.exp(sh), axis=1, keepdims=True))
    out_ref[...] = sh - lse


_f32 = jnp.float32
_tc_first_call = pl.pallas_call(
    _tc_first, out_shape=jax.ShapeDtypeStruct((NP, D), _f32))
_tc_mid_call = pl.pallas_call(
    _tc_mid, out_shape=jax.ShapeDtypeStruct((NP, D), _f32))
_tc_last_call = pl.pallas_call(
    _tc_last, out_shape=jax.ShapeDtypeStruct((N, D), _f32))


def kernel(x, edge_index, W0, b0, W1, b1, W2, b2):
    src = edge_index[0].astype(jnp.int32).reshape(NC, NS, NCH, CH)
    dst = edge_index[1].astype(jnp.int32).reshape(NC, NS, NCH, CH)

    zerosd = jnp.zeros((NP, DW), _f32)
    onesd = jnp.ones((CH, DW), _f32)
    degp = _deg_call(dst, zerosd, onesd)
    p0d = degp[0]
    p1d = degp[1]

    m0 = jax.random.bernoulli(
        jax.random.fold_in(jax.random.key(42), 0), 0.5, (N, D)).astype(jnp.int8)
    m1 = jax.random.bernoulli(
        jax.random.fold_in(jax.random.key(42), 1), 0.5, (N, D)).astype(jnp.int8)

    h0 = _tc_first_call(x, p0d, p1d, W0)
    p = _spmm_call(h0, src, dst)
    h1 = _tc_mid_call(p, h0, p0d, p1d, W1, b0, m0)
    p = _spmm_call(h1, src, dst)
    h2 = _tc_mid_call(p, h1, p0d, p1d, W2, b1, m1)
    p = _spmm_call(h2, src, dst)
    return _tc_last_call(p, h2, p0d, p1d, b2)


# speedup vs baseline: 1.1508x; 1.0007x over previous
import jax
import jax.numpy as jnp
from jax import lax
from jax.experimental import pallas as pl
from jax.experimental.pallas import tpu as pltpu
from jax.experimental.pallas import tpu_sc as plsc

N = 10000
E = 320000
D = 128
NC = 2
NS = 16
NW = NC * NS
EPW = E // NW
CH = 100
NCH = EPW // CH
NP = 10112
RPT = NP // NS
NBUF = 3
ISLOTS = 2 * NBUF
NGRP = NCH // ISLOTS
NTAIL = NCH - NGRP * ISLOTS
DEG_K = 25
DW = 16

_MESH = plsc.VectorSubcoreMesh(core_axis_name="c", subcore_axis_name="s")


def _deg_body(dst_hbm, zero_hbm, ones_hbm, out_hbm, dst_v, ones_v, sem, acc):
    cid = lax.axis_index("c")
    sid = lax.axis_index("s")
    pltpu.sync_copy(dst_hbm.at[cid, sid], dst_v)
    pltpu.sync_copy(ones_hbm, ones_v)
    pltpu.sync_copy(zero_hbm.at[pl.ds(sid * RPT, RPT)], acc.at[pl.ds(sid * RPT, RPT)])
    plsc.subcore_barrier()

    def grp(g, carry):
        def fire(j, c):
            pltpu.async_copy(ones_v, acc.at[dst_v.at[g * DEG_K + j]], sem, add=True)
            return c

        lax.fori_loop(0, DEG_K, fire, 0)

        def drain(j, c):
            pltpu.make_async_copy(ones_v, acc.at[dst_v.at[0]], sem).wait()
            return c

        lax.fori_loop(0, DEG_K, drain, 0)
        return carry

    lax.fori_loop(0, NCH // DEG_K, grp, 0)
    plsc.subcore_barrier()
    pltpu.sync_copy(acc.at[pl.ds(sid * RPT, RPT)],
                    out_hbm.at[cid, pl.ds(sid * RPT, RPT)])


_deg_call = pl.kernel(
    _deg_body,
    out_type=jax.ShapeDtypeStruct((NC, NP, DW), jnp.float32),
    mesh=_MESH,
    scratch_types=[
        pltpu.VMEM((NCH, CH), jnp.int32),
        pltpu.VMEM((CH, DW), jnp.float32),
        pltpu.SemaphoreType.DMA,
        pltpu.VMEM_SHARED((NP, DW), jnp.float32),
    ],
    compiler_params=pltpu.CompilerParams(use_tc_tiling_on_sc=False),
)


def _spmm_body(h_hbm, src_hbm, dst_hbm, out_hbm, rows, sbuf, dbuf,
               gsem, isem, dsem, acc):
    cid = lax.axis_index("c")
    sid = lax.axis_index("s")
    pltpu.sync_copy(h_hbm.at[pl.ds(sid * RPT, RPT)], acc.at[pl.ds(sid * RPT, RPT)])
    plsc.subcore_barrier()

    def fire_idx(j, islot):
        pltpu.async_copy(src_hbm.at[cid, sid, pl.ds(j, 1)],
                         sbuf.at[pl.ds(islot, 1)], isem.at[islot])
        pltpu.async_copy(dst_hbm.at[cid, sid, pl.ds(j, 1)],
                         dbuf.at[pl.ds(islot, 1)], dsem.at[islot])

    def wait_idx(islot, sem):
        pltpu.make_async_copy(src_hbm.at[cid, sid, pl.ds(0, 1)],
                              sbuf.at[pl.ds(islot, 1)], sem.at[islot]).wait()

    def fire_gather(islot, rb):
        pltpu.async_copy(h_hbm.at[sbuf.at[islot]], rows.at[rb], gsem.at[rb])

    def step(j, rb, islot, fire_next):
        if fire_next:
            fire_idx(j + NBUF, (islot + NBUF) % ISLOTS)
        pltpu.make_async_copy(h_hbm.at[sbuf.at[islot]], rows.at[rb],
                              gsem.at[rb]).wait()
        wait_idx(islot, dsem)
        pltpu.sync_copy(rows.at[rb], acc.at[dbuf.at[islot]], add=True)
        if fire_next:
            wait_idx((islot + NBUF) % ISLOTS, isem)
            fire_gather((islot + NBUF) % ISLOTS, rb)

    for b in range(NBUF):
        fire_idx(b, b)
    for b in range(NBUF):
        wait_idx(b, isem)
        fire_gather(b, b)

    def grp(g, carry):
        for s in range(ISLOTS):
            step(g * ISLOTS + s, s % NBUF, s, True)
        return carry

    lax.fori_loop(0, NGRP, grp, 0)
    for t in range(NTAIL):
        j = NGRP * ISLOTS + t
        step(j, t % NBUF, t, t + NBUF < NTAIL)
    plsc.subcore_barrier()
    pltpu.sync_copy(acc.at[pl.ds(sid * RPT, RPT)],
                    out_hbm.at[cid, pl.ds(sid * RPT, RPT)])


_spmm_call = pl.kernel(
    _spmm_body,
    out_type=jax.ShapeDtypeStruct((NC, NP, D), jnp.float32),
    mesh=_MESH,
    scratch_types=[
        pltpu.VMEM((NBUF, CH, D), jnp.float32),
        pltpu.VMEM((ISLOTS, CH), jnp.int32),
        pltpu.VMEM((ISLOTS, CH), jnp.int32),
        pltpu.SemaphoreType.DMA((NBUF,)),
        pltpu.SemaphoreType.DMA((ISLOTS,)),
        pltpu.SemaphoreType.DMA((ISLOTS,)),
        pltpu.VMEM_SHARED((NP, D), jnp.float32),
    ],
)


def _dinv(p0, p1):
    deg = p0[:N, 0] + p1[:N, 0] + 1.0
    return lax.rsqrt(deg)


def _tc_first(x_ref, p0_ref, p1_ref, w_ref, h_out):
    x = x_ref[...]
    mean = jnp.mean(x, axis=0)
    xc = x - mean
    var = jnp.mean(xc * xc, axis=0)
    xn = xc * lax.rsqrt(var + 1e-5)
    s = jnp.sign(xn)
    h = jnp.dot(s, w_ref[...], preferred_element_type=jnp.float32)
    dinv = _dinv(p0_ref[...], p1_ref[...])
    h_out[0:N, :] = h * dinv[:, None]
    h_out[N:NP, :] = jnp.zeros((NP - N, D), jnp.float32)


def _tc_mid(p_ref, hprev_ref, p0_ref, p1_ref, w_ref, b_ref, m2_ref, h_out):
    dinv = _dinv(p0_ref[...], p1_ref[...])
    y = p_ref[0, 0:N, :] + p_ref[1, 0:N, :] - hprev_ref[0:N, :]
    xcur = y * dinv[:, None] + b_ref[...][None, :]
    xcur = xcur * (m2_ref[...].astype(jnp.float32) * 2.0)
    s = jnp.sign(xcur)
    h = jnp.dot(s, w_ref[...], preferred_element_type=jnp.float32)
    h_out[0:N, :] = h * dinv[:, None]
    h_out[N:NP, :] = jnp.zeros((NP - N, D), jnp.float32)


def _tc_last(p_ref, hprev_ref, p0_ref, p1_ref, b_ref, out_ref):
    dinv = _dinv(p0_ref[...], p1_ref[...])
    y = p_ref[0, 0:N, :] + p_ref[1, 0:N, :] - hprev_ref[0:N, :]
    xcur = y * dinv[:, None] + b_ref[...][None, :]
    mx = jnp.max(xcur, axis=1, keepdims=True)
    sh = xcur - mx
    lse = jnp.log(jnp.sum(jnp.exp(sh), axis=1, keepdims=True))
    out_ref[...] = sh - lse


_f32 = jnp.float32
_tc_first_call = pl.pallas_call(
    _tc_first, out_shape=jax.ShapeDtypeStruct((NP, D), _f32))
_tc_mid_call = pl.pallas_call(
    _tc_mid, out_shape=jax.ShapeDtypeStruct((NP, D), _f32))
_tc_last_call = pl.pallas_call(
    _tc_last, out_shape=jax.ShapeDtypeStruct((N, D), _f32))


def kernel(x, edge_index, W0, b0, W1, b1, W2, b2):
    src = edge_index[0].astype(jnp.int32).reshape(NC, NS, NCH, CH)
    dst = edge_index[1].astype(jnp.int32).reshape(NC, NS, NCH, CH)

    zerosd = jnp.zeros((NP, DW), _f32)
    onesd = jnp.ones((CH, DW), _f32)
    degp = _deg_call(dst, zerosd, onesd)
    p0d = degp[0]
    p1d = degp[1]

    m0 = jax.random.bernoulli(
        jax.random.fold_in(jax.random.key(42), 0), 0.5, (N, D)).astype(jnp.int8)
    m1 = jax.random.bernoulli(
        jax.random.fold_in(jax.random.key(42), 1), 0.5, (N, D)).astype(jnp.int8)

    h0 = _tc_first_call(x, p0d, p1d, W0)
    p = _spmm_call(h0, src, dst)
    h1 = _tc_mid_call(p, h0, p0d, p1d, W1, b0, m0)
    p = _spmm_call(h1, src, dst)
    h2 = _tc_mid_call(p, h1, p0d, p1d, W2, b1, m1)
    p = _spmm_call(h2, src, dst)
    return _tc_last_call(p, h2, p0d, p1d, b2)
